# Initial kernel scaffold; baseline (speedup 1.0000x reference)
import jax
import jax.numpy as jnp
from jax.experimental import pallas as pl


def kernel(qk_dots, W1, b1, W2, b2, W3, b3):
    raise NotImplementedError("write your pallas kernel here")


# speedup vs baseline: 32.3634x; 32.3634x over previous
import jax
import jax.numpy as jnp
from jax import lax
from jax.experimental import pallas as pl
import jax.experimental.pallas.tpu as pltpu

N = 2048
H = 16
DIM = 64
LPAD = 4096
TM = 128
W = TM + N


def _table_body(w1_ref, b1_ref, w2t_ref, b2_ref, w3t_ref, b3_ref, out_ref):
    k = lax.broadcasted_iota(jnp.int32, (1, LPAD), 1)
    p = ((N - 1) - k).astype(jnp.float32)
    h1 = jax.nn.relu(w1_ref[...] * p + b1_ref[...])
    h2 = jax.nn.relu(
        jnp.dot(w2t_ref[...], h1, preferred_element_type=jnp.float32)
        + b2_ref[...])
    out_ref[:, 0, :] = (
        jnp.dot(w3t_ref[...], h2, preferred_element_type=jnp.float32)
        + b3_ref[...])


def _bias_add_body(tab_ref, qk_ref, out_ref):
    i = pl.program_id(1)
    start = pl.multiple_of((N - TM) - i * TM, TM)
    w = tab_ref[0, :, pl.ds(start, W)]
    a = jnp.broadcast_to(w, (TM, W))
    riota = lax.broadcasted_iota(jnp.int32, (TM, 1), 0)
    s = (TM - 1) - riota
    for kbit in range(TM.bit_length() - 1):
        sh = 1 << kbit
        rolled = jnp.concatenate([a[:, sh:], a[:, :sh]], axis=1)
        a = jnp.where((s & sh) != 0, rolled, a)
    bias = a[:, :N]
    out_ref[...] = qk_ref[...] + bias[None, None, :, :]


def kernel(qk_dots, W1, b1, W2, b2, W3, b3):
    B = qk_dots.shape[0]
    f32 = jnp.float32

    tabT = pl.pallas_call(
        _table_body,
        out_shape=jax.ShapeDtypeStruct((H, 1, LPAD), f32),
    )(
        W1.T.astype(f32),
        b1.reshape(DIM, 1).astype(f32),
        W2.T.astype(f32),
        b2.reshape(DIM, 1).astype(f32),
        W3.T.astype(f32),
        b3.reshape(H, 1).astype(f32),
    )

    out = pl.pallas_call(
        _bias_add_body,
        grid=(H, N // TM),
        in_specs=[
            pl.BlockSpec((1, 1, LPAD), lambda h, i: (h, 0, 0)),
            pl.BlockSpec((B, 1, TM, N), lambda h, i: (0, h, i, 0)),
        ],
        out_specs=pl.BlockSpec((B, 1, TM, N), lambda h, i: (0, h, i, 0)),
        out_shape=jax.ShapeDtypeStruct(qk_dots.shape, qk_dots.dtype),
        compiler_params=pltpu.CompilerParams(
            dimension_semantics=("parallel", "arbitrary"),
        ),
    )(tabT, qk_dots)
    return out


# speedup vs baseline: 39.3878x; 1.2170x over previous
import jax
import jax.numpy as jnp
from jax import lax
from jax.experimental import pallas as pl
import jax.experimental.pallas.tpu as pltpu

N = 2048
H = 16
DIM = 64
LPAD = 4096
TM = 128
W = TM + N


def _table_body(w1_ref, b1_ref, w2t_ref, b2_ref, w3t_ref, b3_ref, out_ref):
    k = lax.broadcasted_iota(jnp.int32, (1, LPAD), 1)
    p = ((N - 1) - k).astype(jnp.float32)
    h1 = jax.nn.relu(w1_ref[...] * p + b1_ref[...])
    h2 = jax.nn.relu(
        jnp.dot(w2t_ref[...], h1, preferred_element_type=jnp.float32)
        + b2_ref[...])
    out_ref[:, 0, :] = (
        jnp.dot(w3t_ref[...], h2, preferred_element_type=jnp.float32)
        + b3_ref[...])


def _bias_add_body(tab_ref, qk_ref, out_ref):
    i = pl.program_id(1)
    start = pl.multiple_of((N - TM) - i * TM, TM)
    w = tab_ref[0, :, pl.ds(start, W)]
    d = w
    sh = 1
    while sh < TM:
        rot = jnp.concatenate([d[:, sh:], d[:, :sh]], axis=1)
        d = jnp.concatenate([rot, d], axis=0)
        sh *= 2
    bias = d[:, :N]
    out_ref[...] = qk_ref[...] + bias[None, None, :, :]


def kernel(qk_dots, W1, b1, W2, b2, W3, b3):
    B = qk_dots.shape[0]
    f32 = jnp.float32

    tabT = pl.pallas_call(
        _table_body,
        out_shape=jax.ShapeDtypeStruct((H, 1, LPAD), f32),
    )(
        W1.T.astype(f32),
        b1.reshape(DIM, 1).astype(f32),
        W2.T.astype(f32),
        b2.reshape(DIM, 1).astype(f32),
        W3.T.astype(f32),
        b3.reshape(H, 1).astype(f32),
    )

    out = pl.pallas_call(
        _bias_add_body,
        grid=(H, N // TM),
        in_specs=[
            pl.BlockSpec((1, 1, LPAD), lambda h, i: (h, 0, 0)),
            pl.BlockSpec((B, 1, TM, N), lambda h, i: (0, h, i, 0)),
        ],
        out_specs=pl.BlockSpec((B, 1, TM, N), lambda h, i: (0, h, i, 0)),
        out_shape=jax.ShapeDtypeStruct(qk_dots.shape, qk_dots.dtype),
        compiler_params=pltpu.CompilerParams(
            dimension_semantics=("parallel", "arbitrary"),
        ),
    )(tabT, qk_dots)
    return out


# speedup vs baseline: 46.9824x; 1.1928x over previous
import jax
import jax.numpy as jnp
from jax import lax
from jax.experimental import pallas as pl
import jax.experimental.pallas.tpu as pltpu

N = 2048
H = 16
DIM = 64
LPAD = 4096
TM = 256
W = TM + N


def _table_body(w1_ref, b1_ref, w2t_ref, b2_ref, w3t_ref, b3_ref, out_ref):
    k = lax.broadcasted_iota(jnp.int32, (1, LPAD), 1)
    p = ((N - 1) - k).astype(jnp.float32)
    h1 = jax.nn.relu(w1_ref[...] * p + b1_ref[...])
    h2 = jax.nn.relu(
        jnp.dot(w2t_ref[...], h1, preferred_element_type=jnp.float32)
        + b2_ref[...])
    out_ref[:, 0, :] = (
        jnp.dot(w3t_ref[...], h2, preferred_element_type=jnp.float32)
        + b3_ref[...])


def _bias_add_body(tab_ref, qk_ref, out_ref):
    i = pl.program_id(1)
    start = pl.multiple_of((N - TM) - i * TM, TM)
    w = tab_ref[0, :, pl.ds(start, W)]
    d = w
    sh = 1
    while sh < TM:
        rot = jnp.concatenate([d[:, sh:], d[:, :sh]], axis=1)
        d = jnp.concatenate([rot, d], axis=0)
        sh *= 2
    bias = d[:, :N]
    out_ref[...] = qk_ref[...] + bias[None, None, :, :]


def kernel(qk_dots, W1, b1, W2, b2, W3, b3):
    B = qk_dots.shape[0]
    f32 = jnp.float32

    tabT = pl.pallas_call(
        _table_body,
        out_shape=jax.ShapeDtypeStruct((H, 1, LPAD), f32),
    )(
        W1.T.astype(f32),
        b1.reshape(DIM, 1).astype(f32),
        W2.T.astype(f32),
        b2.reshape(DIM, 1).astype(f32),
        W3.T.astype(f32),
        b3.reshape(H, 1).astype(f32),
    )

    out = pl.pallas_call(
        _bias_add_body,
        grid=(H, N // TM),
        in_specs=[
            pl.BlockSpec((1, 1, LPAD), lambda h, i: (h, 0, 0)),
            pl.BlockSpec((B, 1, TM, N), lambda h, i: (0, h, i, 0)),
        ],
        out_specs=pl.BlockSpec((B, 1, TM, N), lambda h, i: (0, h, i, 0)),
        out_shape=jax.ShapeDtypeStruct(qk_dots.shape, qk_dots.dtype),
        compiler_params=pltpu.CompilerParams(
            dimension_semantics=("parallel", "arbitrary"),
        ),
    )(tabT, qk_dots)
    return out


# speedup vs baseline: 47.8759x; 1.0190x over previous
import jax
import jax.numpy as jnp
from jax import lax
from jax.experimental import pallas as pl
import jax.experimental.pallas.tpu as pltpu

N = 2048
H = 16
DIM = 64
LPAD = 4096
TM = 512
W = TM + N


def _table_body(w1_ref, b1_ref, w2t_ref, b2_ref, w3t_ref, b3_ref, out_ref):
    k = lax.broadcasted_iota(jnp.int32, (1, LPAD), 1)
    p = ((N - 1) - k).astype(jnp.float32)
    h1 = jax.nn.relu(w1_ref[...] * p + b1_ref[...])
    h2 = jax.nn.relu(
        jnp.dot(w2t_ref[...], h1, preferred_element_type=jnp.float32)
        + b2_ref[...])
    out_ref[:, 0, :] = (
        jnp.dot(w3t_ref[...], h2, preferred_element_type=jnp.float32)
        + b3_ref[...])


def _bias_add_body(tab_ref, qk_ref, out_ref):
    i = pl.program_id(1)
    start = pl.multiple_of((N - TM) - i * TM, TM)
    w = tab_ref[0, :, pl.ds(start, W)]
    d = w
    sh = 1
    while sh < TM:
        rot = jnp.concatenate([d[:, sh:], d[:, :sh]], axis=1)
        d = jnp.concatenate([rot, d], axis=0)
        sh *= 2
    bias = d[:, :N]
    out_ref[...] = qk_ref[...] + bias[None, None, :, :]


def kernel(qk_dots, W1, b1, W2, b2, W3, b3):
    B = qk_dots.shape[0]
    f32 = jnp.float32

    tabT = pl.pallas_call(
        _table_body,
        out_shape=jax.ShapeDtypeStruct((H, 1, LPAD), f32),
    )(
        W1.T.astype(f32),
        b1.reshape(DIM, 1).astype(f32),
        W2.T.astype(f32),
        b2.reshape(DIM, 1).astype(f32),
        W3.T.astype(f32),
        b3.reshape(H, 1).astype(f32),
    )

    out = pl.pallas_call(
        _bias_add_body,
        grid=(H, N // TM),
        in_specs=[
            pl.BlockSpec((1, 1, LPAD), lambda h, i: (h, 0, 0)),
            pl.BlockSpec((B, 1, TM, N), lambda h, i: (0, h, i, 0)),
        ],
        out_specs=pl.BlockSpec((B, 1, TM, N), lambda h, i: (0, h, i, 0)),
        out_shape=jax.ShapeDtypeStruct(qk_dots.shape, qk_dots.dtype),
        compiler_params=pltpu.CompilerParams(
            dimension_semantics=("parallel", "arbitrary"),
            vmem_limit_bytes=60 * 1024 * 1024,
        ),
    )(tabT, qk_dots)
    return out
